# baseline (speedup 1.0000x reference)
import functools

import jax
import jax.numpy as jnp
from jax import lax
from jax.experimental import pallas as pl
from jax.experimental.pallas import tpu as pltpu
from jax.experimental.pallas import tpu_sc as plsc

NX = 64
NSAMP = 1000000
HALF = NSAMP // 2
BATCH = 16384
NUM_CORES = 2
NUM_SUBCORES = 16
NUM_WORKERS = NUM_CORES * NUM_SUBCORES
B_PER_W = BATCH // NUM_WORKERS
CH = 128
NCH = B_PER_W // CH


@functools.partial(
    pl.kernel,
    out_type=jax.ShapeDtypeStruct((BATCH, NX), jnp.float32),
    mesh=plsc.VectorSubcoreMesh(core_axis_name="c", subcore_axis_name="s"),
    scratch_types=[
        pltpu.VMEM((B_PER_W,), jnp.int32),
        pltpu.VMEM((B_PER_W,), jnp.int32),
        pltpu.VMEM((B_PER_W,), jnp.int32),
        pltpu.VMEM((B_PER_W, NX), jnp.float32),
        pltpu.VMEM((B_PER_W, NX), jnp.float32),
        pltpu.SemaphoreType.DMA,
    ],
    compiler_params=pltpu.CompilerParams(use_tc_tiling_on_sc=False),
)
def _sc_gather(ids_hbm, ta_hbm, tb_hbm, out_hbm, ids_v, ia_v, ib_v, ra_v,
               rb_v, sem):
    wid = lax.axis_index("s") * NUM_CORES + lax.axis_index("c")
    base = wid * B_PER_W
    pltpu.sync_copy(ids_hbm.at[wid], ids_v)
    for s in range(B_PER_W // 16):
        sl = pl.ds(s * 16, 16)
        v = ids_v[sl]
        ia_v[sl] = jnp.minimum(v, HALF - 1)
        ib_v[sl] = jnp.maximum(v - HALF, 0)

    copies = []
    for j in range(NCH):
        sl = pl.ds(j * CH, CH)
        copies.append(
            pltpu.async_copy(ta_hbm.at[ia_v.at[sl]], ra_v.at[sl], sem)
        )
        copies.append(
            pltpu.async_copy(tb_hbm.at[ib_v.at[sl]], rb_v.at[sl], sem)
        )
    for c in copies:
        c.wait()

    def pick(s, carry):
        vec = ids_v[pl.ds(s * 16, 16)]
        for l in range(16):
            j = s * 16 + l
            hi = vec[l] >= HALF
            for k in range(NX // 16):
                kl = pl.ds(k * 16, 16)
                ra_v[j, kl] = jnp.where(hi, rb_v[j, kl], ra_v[j, kl])
        return carry

    lax.fori_loop(0, B_PER_W // 16, pick, 0)

    pltpu.sync_copy(ra_v, out_hbm.at[pl.ds(base, B_PER_W)])


def kernel(ids, start_state):
    ids2 = ids.astype(jnp.int32).reshape(NUM_WORKERS, B_PER_W)
    return _sc_gather(ids2, start_state[:HALF], start_state[HALF:])


# speedup vs baseline: 2.9106x; 2.9106x over previous
import functools

import jax
import jax.numpy as jnp
from jax import lax
from jax.experimental import pallas as pl
from jax.experimental.pallas import tpu as pltpu
from jax.experimental.pallas import tpu_sc as plsc

NX = 64
BATCH = 16384
NUM_CORES = 2
NUM_SUBCORES = 16
NUM_WORKERS = NUM_CORES * NUM_SUBCORES
B_PER_W = BATCH // NUM_WORKERS


@functools.partial(
    pl.kernel,
    out_type=jax.ShapeDtypeStruct((BATCH, NX), jnp.float32),
    mesh=plsc.VectorSubcoreMesh(core_axis_name="c", subcore_axis_name="s"),
    scratch_types=[
        pltpu.VMEM((B_PER_W,), jnp.int32),
        pltpu.VMEM((B_PER_W, NX), jnp.float32),
        pltpu.SemaphoreType.DMA,
    ],
    compiler_params=pltpu.CompilerParams(use_tc_tiling_on_sc=True),
)
def _sc_gather(ids_hbm, table_hbm, out_hbm, ids_v, rows_v, sem):
    wid = lax.axis_index("s") * NUM_CORES + lax.axis_index("c")
    base = wid * B_PER_W
    pltpu.sync_copy(ids_hbm.at[wid], ids_v)

    def issue(s, carry):
        vec = ids_v[pl.ds(s * 16, 16)]
        for l in range(16):
            r = vec[l]
            pltpu.make_async_copy(
                table_hbm.at[r], rows_v.at[s * 16 + l], sem
            ).start()
        return carry

    lax.fori_loop(0, B_PER_W // 16, issue, 0)

    def drain(j, carry):
        pltpu.make_async_copy(table_hbm.at[0], rows_v.at[j], sem).wait()
        return carry

    lax.fori_loop(0, B_PER_W, drain, 0)

    pltpu.sync_copy(rows_v, out_hbm.at[pl.ds(base, B_PER_W)])


def kernel(ids, start_state):
    ids2 = ids.astype(jnp.int32).reshape(NUM_WORKERS, B_PER_W)
    return _sc_gather(ids2, start_state)
